# Initial kernel scaffold; baseline (speedup 1.0000x reference)
import jax
import jax.numpy as jnp
from jax.experimental import pallas as pl


def kernel(center_words, target_words, outer_words, embedding_v, embedding_u):
    raise NotImplementedError("write your pallas kernel here")


# speedup vs baseline: 6.9826x; 6.9826x over previous
import functools

import jax
import jax.numpy as jnp
from jax import lax
from jax.experimental import pallas as pl
from jax.experimental.pallas import tpu as pltpu
from jax.experimental.pallas import tpu_sc as plsc

NC = 2
NS = 16
L = 16


def _sc_partials(cidx, tidx, oidx2, emb_v, emb_u, B, NEG, D):
    NW = NC * NS
    BPW = B // NW
    H = NEG // 2
    NCH = -(-NEG // L)

    mesh = plsc.VectorSubcoreMesh(core_axis_name="c", subcore_axis_name="s")

    @functools.partial(
        pl.kernel,
        out_type=[
            jax.ShapeDtypeStruct((B * L,), jnp.float32),
            jax.ShapeDtypeStruct((B * L,), jnp.float32),
        ],
        mesh=mesh,
        compiler_params=pltpu.CompilerParams(
            use_tc_tiling_on_sc=False, needs_layout_passes=False),
        scratch_types=[
            pltpu.VMEM((BPW,), jnp.int32),
            pltpu.VMEM((BPW,), jnp.int32),
            pltpu.VMEM((2 * BPW, H), jnp.int32),
            pltpu.VMEM((BPW, D), jnp.float32),
            pltpu.VMEM((BPW, D), jnp.float32),
            pltpu.VMEM((NEG, D), jnp.float32),
            pltpu.VMEM((NEG, D), jnp.float32),
            pltpu.VMEM((BPW * L,), jnp.float32),
            pltpu.VMEM((BPW * L,), jnp.float32),
            pltpu.SemaphoreType.DMA,
            pltpu.SemaphoreType.DMA,
            pltpu.SemaphoreType.DMA,
        ],
    )
    def sc_kernel(cidx_hbm, tidx_hbm, oidx_hbm, v_hbm, u_hbm,
                  sp_hbm, ae_hbm,
                  cidx_v, tidx_v, oidx_v, crows, trows, rows0, rows1,
                  sp_v, ae_v, sem0, semA, semB):
        wid = lax.axis_index("s") * NC + lax.axis_index("c")
        b0 = wid * BPW

        pltpu.sync_copy(cidx_hbm.at[pl.ds(b0, BPW)], cidx_v)
        pltpu.sync_copy(tidx_hbm.at[pl.ds(b0, BPW)], tidx_v)
        pltpu.sync_copy(oidx_hbm.at[pl.ds(2 * b0, 2 * BPW)], oidx_v)

        c1 = pltpu.async_copy(v_hbm.at[cidx_v], crows, sem0)
        c2 = pltpu.async_copy(u_hbm.at[tidx_v], trows, sem0)

        iota = lax.iota(jnp.int32, L)
        row_ids = [jnp.minimum(c * L + iota, NEG - 1) for c in range(NCH)]
        valids = [(c * L + iota) < NEG for c in range(NCH)]
        bufs = ((rows0, semA), (rows1, semB))

        def issue(b, rows, sem):
            pltpu.async_copy(u_hbm.at[oidx_v.at[2 * b]],
                             rows.at[pl.ds(0, H)], sem)
            pltpu.async_copy(u_hbm.at[oidx_v.at[2 * b + 1]],
                             rows.at[pl.ds(H, H)], sem)

        def drain(b, rows, sem):
            pltpu.make_async_copy(u_hbm.at[oidx_v.at[2 * b]],
                                  rows.at[pl.ds(0, H)], sem).wait()
            pltpu.make_async_copy(u_hbm.at[oidx_v.at[2 * b + 1]],
                                  rows.at[pl.ds(H, H)], sem).wait()

        c1.wait()
        c2.wait()
        issue(0, rows0, semA)
        issue(1, rows1, semB)

        def compute(b, rows):
            spv = crows[b, pl.ds(0, L)] * trows[b, pl.ds(0, L)]
            for k in range(1, D // L):
                spv = spv + crows[b, pl.ds(k * L, L)] * trows[b, pl.ds(k * L, L)]
            sp_v[pl.ds(b * L, L)] = spv

            def dstep(d, accs):
                dfull = jnp.full((L,), d, jnp.int32)
                cval = plsc.load_gather(crows, [jnp.full((L,), b, jnp.int32), dfull])
                return tuple(
                    accs[c] + plsc.load_gather(rows, [row_ids[c], dfull]) * cval
                    for c in range(NCH))

            accs = tuple(jnp.zeros((L,), jnp.float32) for _ in range(NCH))
            accs = lax.fori_loop(0, D, dstep, accs)

            aev = jnp.zeros((L,), jnp.float32)
            for c in range(NCH):
                aev = aev + jnp.where(valids[c], jnp.exp(accs[c]), 0.0)
            ae_v[pl.ds(b * L, L)] = aev

        def body(i, carry):
            for p in range(2):
                rows, sem = bufs[p]
                b = 2 * i + p
                drain(b, rows, sem)
                compute(b, rows)

                @pl.when(b + 2 < BPW)
                def _():
                    issue(b + 2, rows, sem)
            return carry

        lax.fori_loop(0, BPW // 2, body, 0)

        pltpu.sync_copy(sp_v, sp_hbm.at[pl.ds(b0 * L, BPW * L)])
        pltpu.sync_copy(ae_v, ae_hbm.at[pl.ds(b0 * L, BPW * L)])

    return sc_kernel(cidx, tidx, oidx2, emb_v, emb_u)


def _tc_reduce(sp, ae, B):
    def body(sp_ref, ae_ref, o_ref):
        scores_total = jnp.sum(sp_ref[...])
        lse_total = jnp.sum(jnp.log(jnp.sum(ae_ref[...], axis=1)))
        o_ref[...] = jnp.reshape(-(scores_total - lse_total) / B, (1, 1))

    return pl.pallas_call(
        body,
        out_shape=jax.ShapeDtypeStruct((1, 1), jnp.float32),
    )(sp, ae)


def kernel(center_words, target_words, outer_words, embedding_v, embedding_u):
    B = center_words.shape[0]
    NEG = outer_words.shape[1]
    D = embedding_v.shape[1]

    cidx = center_words.reshape(B).astype(jnp.int32)
    tidx = target_words.reshape(B).astype(jnp.int32)
    oidx2 = outer_words.astype(jnp.int32).reshape(2 * B, NEG // 2)

    sp, ae = _sc_partials(cidx, tidx, oidx2, embedding_v, embedding_u,
                          B, NEG, D)
    nll = _tc_reduce(sp.reshape(B, L), ae.reshape(B, L), B)
    return nll[0, 0]
